# Initial kernel scaffold; baseline (speedup 1.0000x reference)
import jax
import jax.numpy as jnp
from jax.experimental import pallas as pl


def kernel(item_id, category, brand, title, price, created_at, pk_table, text_table, W1, b1, W2, b2):
    raise NotImplementedError("write your pallas kernel here")


# speedup vs baseline: 5.3406x; 5.3406x over previous
import functools

import jax
import jax.numpy as jnp
from jax import lax
from jax.experimental import pallas as pl
from jax.experimental.pallas import tpu as pltpu
from jax.experimental.pallas import tpu_sc as plsc

B = 16384
PK = 100001
D = 64
TV = 1000
T = 16

NC = 2
NS = 16
NW = NC * NS
BPW = B // NW
PK_CH = 256


def _sc_gather(item_id, title, pk_table, text_flat):
    mesh = plsc.VectorSubcoreMesh(core_axis_name="c", subcore_axis_name="s",
                                  num_cores=NC, num_subcores=NS)

    @functools.partial(
        pl.kernel,
        out_type=[
            jax.ShapeDtypeStruct((B, D), jnp.float32),
            jax.ShapeDtypeStruct((B, D), jnp.float32),
        ],
        mesh=mesh,
        compiler_params=pltpu.CompilerParams(needs_layout_passes=False,
                                             use_tc_tiling_on_sc=False),
        scratch_types=[
            pltpu.VMEM((BPW,), jnp.int32),
            pltpu.VMEM((BPW, T), jnp.int32),
            pltpu.VMEM((TV * D,), jnp.float32),
            pltpu.VMEM((BPW, D), jnp.float32),
            pltpu.SemaphoreType.DMA,
        ],
    )
    def k(item_hbm, title_hbm, pk_hbm, text_hbm, pkrows_hbm, tsum_hbm,
          idx_v, title_v, table_v, buf_v, sem):
        wid = lax.axis_index("s") * NC + lax.axis_index("c")
        base = wid * BPW

        pltpu.sync_copy(text_hbm, table_v)
        pltpu.sync_copy(item_hbm.at[pl.ds(base, BPW)], idx_v)
        pltpu.sync_copy(title_hbm.at[pl.ds(base, BPW)], title_v)

        zero16 = jnp.zeros((16,), jnp.float32)
        for kk in range(D // 16):
            table_v[pl.ds(kk * 16, 16)] = zero16

        pltpu.async_copy(pk_hbm.at[idx_v], buf_v, sem).wait()
        pltpu.sync_copy(buf_v, pkrows_hbm.at[pl.ds(base, BPW)])

        iota = lax.broadcasted_iota(jnp.int32, (16,), 0)

        def row_body(r, carry):
            accs = [jnp.zeros((16,), jnp.float32) for _ in range(D // 16)]
            trow = title_v[r, :] * D
            for t in range(T):
                off = trow[t]
                for kk in range(D // 16):
                    g = plsc.load_gather(table_v, [off + (kk * 16) + iota])
                    accs[kk] = accs[kk] + g
            for kk in range(D // 16):
                buf_v[r, pl.ds(kk * 16, 16)] = accs[kk]
            return carry

        lax.fori_loop(0, BPW, row_body, 0)
        pltpu.sync_copy(buf_v, tsum_hbm.at[pl.ds(base, BPW)])

    return k(item_id, title, pk_table, text_flat)


def _tc_fnn(pk_rows, tsum, title, cat, brand, pt,
            W1pk, W1cat, W1brand, W1text, W1pt, b1, W2, b2):
    BLK = 1024
    grid = (B // BLK,)
    row_spec2 = lambda w: pl.BlockSpec((BLK, w), lambda i: (i, 0))
    full_spec = lambda a, b: pl.BlockSpec((a, b), lambda i: (0, 0))

    def body(pk_ref, tsum_ref, title_ref, cat_ref, brand_ref, pt_ref,
             w1pk_ref, w1c_ref, w1b_ref, w1t_ref, w1pt_ref, b1_ref,
             w2_ref, b2_ref, out_ref):
        hi = jax.lax.Precision.HIGHEST
        ttl = title_ref[...]
        cnt = jnp.sum((ttl != 0).astype(jnp.float32), axis=1, keepdims=True)
        text = tsum_ref[...] / jnp.maximum(cnt, 1.0)
        oh_c = (cat_ref[...] ==
                lax.broadcasted_iota(jnp.int32, (BLK, 16), 1)).astype(
                    jnp.float32)
        oh_b = (brand_ref[...] ==
                lax.broadcasted_iota(jnp.int32, (BLK, 8), 1)).astype(
                    jnp.float32)
        x1 = jnp.dot(pk_ref[...], w1pk_ref[...], precision=hi,
                     preferred_element_type=jnp.float32)
        x1 += jnp.dot(text, w1t_ref[...], precision=hi,
                      preferred_element_type=jnp.float32)
        x1 += jnp.dot(oh_c, w1c_ref[...], precision=hi,
                      preferred_element_type=jnp.float32)
        x1 += jnp.dot(oh_b, w1b_ref[...], precision=hi,
                      preferred_element_type=jnp.float32)
        x1 += jnp.dot(pt_ref[...], w1pt_ref[...], precision=hi,
                      preferred_element_type=jnp.float32)
        h = jnp.maximum(x1 + b1_ref[...], 0.0)
        out_ref[...] = jnp.dot(h, w2_ref[...], precision=hi,
                               preferred_element_type=jnp.float32) + b2_ref[...]

    return pl.pallas_call(
        body,
        grid=grid,
        in_specs=[
            row_spec2(D),
            row_spec2(D),
            row_spec2(T),
            row_spec2(1),
            row_spec2(1),
            row_spec2(2),
            full_spec(D, D),
            full_spec(16, D),
            full_spec(8, D),
            full_spec(D, D),
            full_spec(2, D),
            full_spec(1, D),
            full_spec(D, D),
            full_spec(1, D),
        ],
        out_specs=row_spec2(D),
        out_shape=jax.ShapeDtypeStruct((B, D), jnp.float32),
    )(pk_rows, tsum, title, cat, brand, pt,
      W1pk, W1cat, W1brand, W1text, W1pt, b1, W2, b2)


def kernel(item_id, category, brand, title, price, created_at,
           pk_table, text_table, W1, b1, W2, b2):
    item_id = item_id.astype(jnp.int32)
    title_i = title.astype(jnp.int32)
    pk_rows, tsum = _sc_gather(item_id, title_i, pk_table,
                               text_table.reshape(-1))
    pt = jnp.stack([price, created_at], axis=1)
    out = _tc_fnn(
        pk_rows, tsum, title_i,
        category.astype(jnp.int32).reshape(B, 1),
        brand.astype(jnp.int32).reshape(B, 1),
        pt,
        W1[0:D],
        W1[D:D + 16],
        W1[D + 16:D + 24],
        W1[D + 24:D + 24 + D],
        W1[D + 24 + D:],
        b1.reshape(1, D),
        W2,
        b2.reshape(1, D),
    )
    return out
